# Initial kernel scaffold; baseline (speedup 1.0000x reference)
import jax
import jax.numpy as jnp
from jax.experimental import pallas as pl


def kernel(x):
    raise NotImplementedError("write your pallas kernel here")


# speedup vs baseline: 5.5785x; 5.5785x over previous
import jax
import jax.numpy as jnp
from jax.experimental import pallas as pl
from jax.experimental.pallas import tpu as pltpu

D = 64
N_FEAT = 2048
CHUNK = 256
K_SIGN = 34
K_SQRT = 22
EPS = 1e-12


def _pool_body(x_ref, m_ref):
    def chunk_max(i, m):
        xc = x_ref[0, pl.ds(i * CHUNK, CHUNK), :]
        p = xc[:, :, None] * xc[:, None, :]
        return jnp.maximum(m, jnp.max(p, axis=0))

    m0 = jnp.full((D, D), -jnp.inf, dtype=jnp.float32)
    m_ref[0] = jax.lax.fori_loop(0, N_FEAT // CHUNK, chunk_max, m0)


def _eye(n):
    r = jax.lax.broadcasted_iota(jnp.int32, (n, n), 0)
    c = jax.lax.broadcasted_iota(jnp.int32, (n, n), 1)
    return (r == c).astype(jnp.float32)


def _mm(a, b):
    return jnp.dot(a, b, preferred_element_type=jnp.float32)


def _sqrt_body(m_ref, o_ref):
    M = m_ref[0]
    I = _eye(D)

    c = jnp.sqrt(jnp.sum(M * M))
    S = M / c

    for _ in range(K_SIGN):
        S2 = _mm(S, S)
        S = _mm(1.5 * I - 0.5 * S2, S)

    A = _mm(S, M)
    s = jnp.sqrt(jnp.sum(A * A))
    Y = A / s
    Z = I

    for _ in range(K_SQRT):
        T = 1.5 * I - 0.5 * _mm(Z, Y)
        Y = _mm(Y, T)
        Z = _mm(T, Z)

    sq = _mm(S, Y) * jnp.sqrt(s)
    n = jnp.sqrt(jnp.sum(sq * sq))
    o_ref[0] = sq / jnp.maximum(n, EPS)


def kernel(x):
    B, T, N, Dd = x.shape
    BT = B * T
    xr = x.reshape(BT, N, Dd)

    m = pl.pallas_call(
        _pool_body,
        grid=(BT,),
        in_specs=[pl.BlockSpec((1, N, Dd), lambda b: (b, 0, 0))],
        out_specs=pl.BlockSpec((1, Dd, Dd), lambda b: (b, 0, 0)),
        out_shape=jax.ShapeDtypeStruct((BT, Dd, Dd), jnp.float32),
        compiler_params=pltpu.CompilerParams(
            dimension_semantics=("parallel",)),
    )(xr)

    v = pl.pallas_call(
        _sqrt_body,
        grid=(BT,),
        in_specs=[pl.BlockSpec((1, Dd, Dd), lambda b: (b, 0, 0))],
        out_specs=pl.BlockSpec((1, Dd, Dd), lambda b: (b, 0, 0)),
        out_shape=jax.ShapeDtypeStruct((BT, Dd, Dd), jnp.float32),
        compiler_params=pltpu.CompilerParams(
            dimension_semantics=("parallel",)),
    )(m)

    return jnp.squeeze(v.reshape(B, T, Dd * Dd))


# speedup vs baseline: 44.1183x; 7.9086x over previous
import jax
import jax.numpy as jnp
from jax.experimental import pallas as pl
from jax.experimental.pallas import tpu as pltpu

D = 64
N_FEAT = 2048
GSUB = 8
K_QUINT = 11
K_CUBIC = 4
K_PADE = 12
QA, QB, QC = 3.4445, -4.7750, 2.0315
EPS = 1e-12


def _pool_body(xt_ref, m_ref):
    Xt = xt_ref[0]
    cols = []
    for i in range(D):
        p = Xt * Xt[i:i + 1, :]
        cols.append(jnp.max(p, axis=1, keepdims=True))
    m_ref[0] = jnp.concatenate(cols, axis=1)


def _eye(n):
    r = jax.lax.broadcasted_iota(jnp.int32, (n, n), 0)
    c = jax.lax.broadcasted_iota(jnp.int32, (n, n), 1)
    return (r == c).astype(jnp.float32)


def _mm(a, b):
    return jnp.dot(a, b, preferred_element_type=jnp.float32)


def _frob(a):
    return jnp.sqrt(jnp.sum(a * a))


def _sqrt_body(m_ref, o_ref):
    I = _eye(D)
    Ms = [m_ref[g] for g in range(GSUB)]
    Ss = [M / _frob(M) for M in Ms]

    for _ in range(K_QUINT):
        S2s = [_mm(S, S) for S in Ss]
        S4s = [_mm(S2, S2) for S2 in S2s]
        Ss = [_mm(QA * I + QB * S2 + QC * S4, S)
              for S, S2, S4 in zip(Ss, S2s, S4s)]
    for _ in range(K_CUBIC):
        S2s = [_mm(S, S) for S in Ss]
        Ss = [_mm(1.5 * I - 0.5 * S2, S) for S, S2 in zip(Ss, S2s)]

    As = [_mm(S, M) for S, M in zip(Ss, Ms)]
    ss = [_frob(A) for A in As]
    Ys = [A / s for A, s in zip(As, ss)]
    Zs = [I for _ in range(GSUB)]

    for _ in range(K_PADE):
        Ws = [_mm(Z, Y) for Z, Y in zip(Zs, Ys)]
        W2s = [_mm(W, W) for W in Ws]
        Ts = [(15.0 * I - 10.0 * W + 3.0 * W2) / 8.0
              for W, W2 in zip(Ws, W2s)]
        Ys = [_mm(Y, T) for Y, T in zip(Ys, Ts)]
        Zs = [_mm(T, Z) for T, Z in zip(Ts, Zs)]

    for g in range(GSUB):
        sq = _mm(Ss[g], Ys[g]) * jnp.sqrt(ss[g])
        n = _frob(sq)
        o_ref[g] = sq / jnp.maximum(n, EPS)


def kernel(x):
    B, T, N, Dd = x.shape
    BT = B * T
    xt = jnp.swapaxes(x.reshape(BT, N, Dd), 1, 2)

    m = pl.pallas_call(
        _pool_body,
        grid=(BT,),
        in_specs=[pl.BlockSpec((1, Dd, N), lambda b: (b, 0, 0))],
        out_specs=pl.BlockSpec((1, Dd, Dd), lambda b: (b, 0, 0)),
        out_shape=jax.ShapeDtypeStruct((BT, Dd, Dd), jnp.float32),
        compiler_params=pltpu.CompilerParams(
            dimension_semantics=("parallel",)),
    )(xt)

    v = pl.pallas_call(
        _sqrt_body,
        grid=(BT // GSUB,),
        in_specs=[pl.BlockSpec((GSUB, Dd, Dd), lambda b: (b, 0, 0))],
        out_specs=pl.BlockSpec((GSUB, Dd, Dd), lambda b: (b, 0, 0)),
        out_shape=jax.ShapeDtypeStruct((BT, Dd, Dd), jnp.float32),
        compiler_params=pltpu.CompilerParams(
            dimension_semantics=("parallel",)),
    )(m)

    return jnp.squeeze(v.reshape(B, T, Dd * Dd))
